# Initial kernel scaffold; baseline (speedup 1.0000x reference)
import jax
import jax.numpy as jnp
from jax.experimental import pallas as pl


def kernel(data, indices, x_mark_enc, x_dec, x_mark_dec, Ws, bs):
    raise NotImplementedError("write your pallas kernel here")


# speedup vs baseline: 30.6817x; 30.6817x over previous
import functools

import jax
import jax.numpy as jnp
import numpy as np
from jax.experimental import pallas as pl
from jax.experimental.pallas import tpu as pltpu

SEQ_LEN = 336
PRED_LEN = 96
C_BLOCKS = 4
ZOO = 3
K = 2
BATCH = 32
CH = 128
KERNEL_W = 25
OUT_LEN = PRED_LEN * C_BLOCKS


def _movavg_matrix() -> np.ndarray:
    pad = (KERNEL_W - 1) // 2
    a = np.zeros((SEQ_LEN, SEQ_LEN), dtype=np.float64)
    for t in range(SEQ_LEN):
        for u in range(t - pad, t + pad + 1):
            a[t, min(max(u, 0), SEQ_LEN - 1)] += 1.0 / KERNEL_W
    return a.astype(np.float32)


_A = _movavg_matrix()


def _compose_body(a_ref, w_ref, b_ref, m_ref, d_ref):
    a = a_ref[...]
    eye = (jax.lax.broadcasted_iota(jnp.int32, (SEQ_LEN, SEQ_LEN), 0)
           == jax.lax.broadcasted_iota(jnp.int32, (SEQ_LEN, SEQ_LEN), 1)
           ).astype(jnp.float32)
    cur_m = eye
    cur_b = jnp.zeros((SEQ_LEN, 1), dtype=jnp.float32)
    for blk in range(C_BLOCKS):
        wse = w_ref[0, blk, 0]
        wtr = w_ref[0, blk, 1]
        bsum = b_ref[0, blk, 0] + b_ref[0, blk, 1]
        d = wse + jnp.dot(wtr - wse, a, preferred_element_type=jnp.float32)
        m_blk = jnp.dot(d, cur_m, preferred_element_type=jnp.float32)
        b_blk = jnp.dot(d, cur_b, preferred_element_type=jnp.float32) + bsum
        m_ref[0, blk * PRED_LEN:(blk + 1) * PRED_LEN, :] = m_blk
        d_ref[0, blk * PRED_LEN:(blk + 1) * PRED_LEN, :] = b_blk
        cur_m = jnp.concatenate([cur_m[PRED_LEN:], m_blk], axis=0)
        cur_b = jnp.concatenate([cur_b[PRED_LEN:], b_blk], axis=0)


def _apply_body(idx_ref, data_ref, m_ref, d_ref, out_ref):
    b = pl.program_id(0)
    x = data_ref[0]
    mean = jnp.mean(x, axis=0, keepdims=True)
    xc = x - mean
    var = jnp.mean(xc * xc, axis=0, keepdims=True)
    stdev = jnp.sqrt(var + 1e-5)
    xn = xc / stdev
    e0 = idx_ref[0, b]
    e1 = idx_ref[1, b]
    w = [0.5 * ((e0 == i).astype(jnp.float32) + (e1 == i).astype(jnp.float32))
         for i in range(ZOO)]
    mb = w[0] * m_ref[0] + w[1] * m_ref[1] + w[2] * m_ref[2]
    db = w[0] * d_ref[0] + w[1] * d_ref[1] + w[2] * d_ref[2]
    y = jnp.dot(mb, xn, preferred_element_type=jnp.float32) + db
    out_ref[0] = y * stdev + mean


@functools.partial(jax.jit, static_argnames=("interpret",))
def _run(data, indices, ws, bs, interpret=False):
    a = jnp.asarray(_A)
    bs_col = bs.reshape(ZOO, C_BLOCKS, 2, PRED_LEN, 1)
    m, d = pl.pallas_call(
        _compose_body,
        grid=(ZOO,),
        in_specs=[
            pl.BlockSpec((SEQ_LEN, SEQ_LEN), lambda i: (0, 0)),
            pl.BlockSpec((1, C_BLOCKS, 2, PRED_LEN, SEQ_LEN),
                         lambda i: (i, 0, 0, 0, 0)),
            pl.BlockSpec((1, C_BLOCKS, 2, PRED_LEN, 1),
                         lambda i: (i, 0, 0, 0, 0)),
        ],
        out_specs=[
            pl.BlockSpec((1, OUT_LEN, SEQ_LEN), lambda i: (i, 0, 0)),
            pl.BlockSpec((1, OUT_LEN, 1), lambda i: (i, 0, 0)),
        ],
        out_shape=[
            jax.ShapeDtypeStruct((ZOO, OUT_LEN, SEQ_LEN), jnp.float32),
            jax.ShapeDtypeStruct((ZOO, OUT_LEN, 1), jnp.float32),
        ],
        interpret=interpret,
    )(a, ws, bs_col)

    out = pl.pallas_call(
        _apply_body,
        grid=(BATCH,),
        in_specs=[
            pl.BlockSpec(memory_space=pltpu.SMEM),
            pl.BlockSpec((1, SEQ_LEN, CH), lambda b: (b, 0, 0)),
            pl.BlockSpec((ZOO, OUT_LEN, SEQ_LEN), lambda b: (0, 0, 0)),
            pl.BlockSpec((ZOO, OUT_LEN, 1), lambda b: (0, 0, 0)),
        ],
        out_specs=pl.BlockSpec((1, OUT_LEN, CH), lambda b: (b, 0, 0)),
        out_shape=jax.ShapeDtypeStruct((BATCH, OUT_LEN, CH), jnp.float32),
        interpret=interpret,
    )(indices, data, m, d)
    return out


def kernel(data, indices, x_mark_enc, x_dec, x_mark_dec, Ws, bs):
    return _run(data, indices.astype(jnp.int32), Ws, bs)
